# baseline (speedup 1.0000x reference)
import functools

import jax
import jax.numpy as jnp
from jax import lax
from jax.experimental import pallas as pl
from jax.experimental.pallas import tpu as pltpu
from jax.experimental.pallas import tpu_sc as plsc

N_NODES = 10000
N_EDGES = 320000
D = 128
ED = 16

NC = 2
NS = 16
NW = NC * NS
EP = N_EDGES // NW
K = 80
CHUNKS = EP // K
RING = 3
IDXG = 25
NGRP = CHUNKS // IDXG
SUB_ROWS = 624
TAIL_ROWS = N_NODES - NS * SUB_ROWS

_mesh = functools.partial(
    plsc.VectorSubcoreMesh, core_axis_name="c", subcore_axis_name="s")


def _pre_body(x_ref, w_ref, a_ref, b_ref):
    ab = jnp.dot(x_ref[...], w_ref[...], preferred_element_type=jnp.float32)
    a_ref[...] = ab[:, :D]
    b_ref[...] = ab[:, D:]


def _precompute(x, w_ab):
    return pl.pallas_call(
        _pre_body,
        out_shape=(jax.ShapeDtypeStruct((N_NODES, D), jnp.float32),
                   jax.ShapeDtypeStruct((N_NODES, D), jnp.float32)),
    )(x, w_ab)


def _gather_body(a_hbm, b_hbm, packed_idx, g_hbm,
                 idxb, ra, rb, gb, sem_a, sem_b, sem_w):
    c = lax.axis_index("c")
    s = lax.axis_index("s")
    wid = s * NC + c

    pltpu.sync_copy(packed_idx.at[wid * NGRP], idxb.at[0])

    def gather_descs(j, slot):
        g2 = j // IDXG
        jj2 = j % IDXG
        da = pltpu.make_async_copy(
            a_hbm.at[idxb.at[g2 % 2, 2 * jj2, pl.ds(0, K)]],
            ra.at[slot], sem_a.at[slot])
        db = pltpu.make_async_copy(
            b_hbm.at[idxb.at[g2 % 2, 2 * jj2 + 1, pl.ds(0, K)]],
            rb.at[slot], sem_b.at[slot])
        return da, db

    def issue(j, slot):
        da, db = gather_descs(j, slot)
        da.start()
        db.start()

    def drain_write(slot):
        pltpu.make_async_copy(gb.at[slot], g_hbm.at[pl.ds(0, K)],
                              sem_w.at[slot]).wait()

    issue(0, 0)
    issue(1, 1)

    def chunk(j, carry):
        slot = j % RING
        g = j // IDXG
        jj = j % IDXG
        da, db = gather_descs(j, slot)
        da.wait()
        db.wait()

        @pl.when((jj == IDXG - 2) & (g + 1 < NGRP))
        def _stage_idx():
            pltpu.sync_copy(packed_idx.at[wid * NGRP + g + 1],
                            idxb.at[(g + 1) % 2])

        @pl.when(j >= RING)
        def _drain():
            drain_write(slot)

        def add_row(e, carry2):
            for v in range(D // 16):
                sl = pl.ds(v * 16, 16)
                gb[slot, e, sl] = ra[slot, e, sl] + rb[slot, e, sl]
            return carry2

        lax.fori_loop(0, K, add_row, 0, unroll=False)
        off = pl.multiple_of(wid * EP + j * K, 8)
        pltpu.async_copy(gb.at[slot], g_hbm.at[pl.ds(off, K)], sem_w.at[slot])

        @pl.when(j + 2 < CHUNKS)
        def _prefetch():
            issue(j + 2, (j + 2) % RING)

        return carry

    lax.fori_loop(0, CHUNKS, chunk, 0, unroll=False)
    for jj in range(CHUNKS - RING, CHUNKS):
        drain_write(jj % RING)


def _gather_add(a, b, packed_idx):
    return pl.kernel(
        _gather_body,
        out_type=jax.ShapeDtypeStruct((N_EDGES, D), jnp.float32),
        mesh=_mesh(),
        scratch_types=[
            pltpu.VMEM((2, 2 * IDXG, 128), jnp.int32),
            pltpu.VMEM((RING, K, D), jnp.float32),
            pltpu.VMEM((RING, K, D), jnp.float32),
            pltpu.VMEM((RING, K, D), jnp.float32),
            pltpu.SemaphoreType.DMA((RING,)),
            pltpu.SemaphoreType.DMA((RING,)),
            pltpu.SemaphoreType.DMA((RING,)),
        ],
    )(a, b, packed_idx)


def _edge_mlp_body(g_ref, ea_ref, w1c_ref, b1_ref, w2_ref, b2_ref, m_ref):
    z = (g_ref[...]
         + jnp.dot(ea_ref[...], w1c_ref[...], preferred_element_type=jnp.float32)
         + b1_ref[...])
    h = jnp.maximum(z, 0.0).astype(jnp.bfloat16)
    m_ref[...] = (jnp.dot(h, w2_ref[...], preferred_element_type=jnp.float32)
                  + b2_ref[...])


def _edge_mlp(g, edge_attr, w1c, b1, w2, b2, block_e=4000):
    ne = g.shape[0]
    grid = ne // block_e
    return pl.pallas_call(
        _edge_mlp_body,
        grid=(grid,),
        in_specs=[
            pl.BlockSpec((block_e, D), lambda i: (i, 0)),
            pl.BlockSpec((block_e, ED), lambda i: (i, 0)),
            pl.BlockSpec((ED, D), lambda i: (0, 0)),
            pl.BlockSpec((1, D), lambda i: (0, 0)),
            pl.BlockSpec((D, D), lambda i: (0, 0)),
            pl.BlockSpec((1, D), lambda i: (0, 0)),
        ],
        out_specs=pl.BlockSpec((block_e, D), lambda i: (i, 0)),
        out_shape=jax.ShapeDtypeStruct((ne, D), jnp.float32),
    )(g, edge_attr, w1c, b1, w2, b2)


def _scatter_body(m_hbm, row3d, part_hbm, idx_r, mb, agg, sem_l):
    c = lax.axis_index("c")
    s = lax.axis_index("s")
    wid = s * NC + c

    def zrow(e, carry):
        for v in range(D // 16):
            mb[0, e, pl.ds(v * 16, 16)] = jnp.zeros((16,), jnp.float32)
        return carry

    lax.fori_loop(0, K, zrow, 0, unroll=False)
    for t in range(7):
        zoff = pl.multiple_of(s * SUB_ROWS + t * K, 8)
        pltpu.sync_copy(mb.at[0], agg.at[pl.ds(zoff, K)])
    zoff = pl.multiple_of(s * SUB_ROWS + 7 * K, 8)
    pltpu.sync_copy(mb.at[0, pl.ds(0, 64)], agg.at[pl.ds(zoff, 64)])

    @pl.when(s == NS - 1)
    def _zero_tail():
        pltpu.sync_copy(mb.at[0, pl.ds(0, TAIL_ROWS)],
                        agg.at[pl.ds(NS * SUB_ROWS, TAIL_ROWS)])

    plsc.subcore_barrier()

    pltpu.sync_copy(row3d.at[wid], idx_r)

    def load(j, slot):
        off = pl.multiple_of(wid * EP + j * K, 8)
        pltpu.async_copy(m_hbm.at[pl.ds(off, K)], mb.at[slot], sem_l.at[slot])

    load(0, 0)

    def chunk(j, carry):
        slot = j % 2
        pltpu.make_async_copy(m_hbm.at[pl.ds(0, K)], mb.at[slot],
                              sem_l.at[slot]).wait()

        @pl.when(j + 1 < CHUNKS)
        def _prefetch():
            load(j + 1, (j + 1) % 2)

        pltpu.sync_copy(mb.at[slot], agg.at[idx_r.at[j]], add=True)
        return carry

    lax.fori_loop(0, CHUNKS, chunk, 0, unroll=False)
    plsc.subcore_barrier()

    woff = pl.multiple_of(s * SUB_ROWS, 8)
    pltpu.sync_copy(agg.at[pl.ds(woff, SUB_ROWS)],
                    part_hbm.at[c, pl.ds(woff, SUB_ROWS)])

    @pl.when(s == NS - 1)
    def _write_tail():
        pltpu.sync_copy(agg.at[pl.ds(NS * SUB_ROWS, TAIL_ROWS)],
                        part_hbm.at[c, pl.ds(NS * SUB_ROWS, TAIL_ROWS)])


def _scatter_add(m, row3d):
    return pl.kernel(
        _scatter_body,
        out_type=jax.ShapeDtypeStruct((NC, N_NODES, D), jnp.float32),
        mesh=_mesh(),
        scratch_types=[
            pltpu.VMEM((CHUNKS, K), jnp.int32),
            pltpu.VMEM((2, K, D), jnp.float32),
            pltpu.VMEM_SHARED((N_NODES, D), jnp.float32),
            pltpu.SemaphoreType.DMA((2,)),
        ],
    )(m, row3d)


def _node_mlp_body(x_ref, p_ref, wnx_ref, wna_ref, bn1_ref, wn2_ref, bn2_ref,
                   o_ref):
    p = p_ref[...]
    agg = p[0] + p[1]
    t = (jnp.dot(x_ref[...], wnx_ref[...], preferred_element_type=jnp.float32)
         + jnp.dot(agg, wna_ref[...], preferred_element_type=jnp.float32)
         + bn1_ref[...])
    h = jnp.maximum(t, 0.0)
    o_ref[...] = (jnp.dot(h, wn2_ref[...], preferred_element_type=jnp.float32)
                  + bn2_ref[...])


def _node_mlp(x, parts, wnx, wna, bn1, wn2, bn2):
    return pl.pallas_call(
        _node_mlp_body,
        out_shape=jax.ShapeDtypeStruct((N_NODES, D), jnp.float32),
    )(x, parts, wnx, wna, bn1, wn2, bn2)


def kernel(x, edge_index, edge_attr, W_e1, b_e1, W_e2, b_e2,
           W_n1, b_n1, W_n2, b_n2):
    row = edge_index[0].astype(jnp.int32)
    col = edge_index[1].astype(jnp.int32)
    row3d = row.reshape(NW, CHUNKS, K)
    zpad = jnp.zeros((NW, CHUNKS, 128 - K), jnp.int32)
    rrows = jnp.concatenate([row.reshape(NW, CHUNKS, K), zpad], axis=2)
    crows = jnp.concatenate([col.reshape(NW, CHUNKS, K), zpad], axis=2)
    packed_idx = jnp.stack([rrows, crows], axis=2).reshape(
        NW * NGRP, 2 * IDXG, 128)

    w_ab = jnp.concatenate([W_e1[:D], W_e1[D:2 * D]], axis=1)
    a, b = _precompute(x, w_ab)

    g = _gather_add(a, b, packed_idx)

    m = _edge_mlp(g, edge_attr, W_e1[2 * D:], b_e1.reshape(1, D),
                  W_e2.astype(jnp.bfloat16), b_e2.reshape(1, D))

    parts = _scatter_add(m, row3d)

    out = _node_mlp(x, parts, W_n1[:D], W_n1[D:], b_n1.reshape(1, D),
                    W_n2, b_n2.reshape(1, D))
    return out


# speedup vs baseline: 1.1974x; 1.1974x over previous
import functools

import jax
import jax.numpy as jnp
from jax import lax
from jax.experimental import pallas as pl
from jax.experimental.pallas import tpu as pltpu
from jax.experimental.pallas import tpu_sc as plsc

N_NODES = 10000
N_EDGES = 320000
D = 128
ED = 16

NC = 2
NS = 16
NW = NC * NS
EP = N_EDGES // NW
K = 80
CHUNKS = EP // K
RING = 3
IDXG = 25
NGRP = CHUNKS // IDXG
SUB_ROWS = 624
TAIL_ROWS = N_NODES - NS * SUB_ROWS

_mesh = functools.partial(
    plsc.VectorSubcoreMesh, core_axis_name="c", subcore_axis_name="s")


def _pre_body(x_ref, w_ref, a_ref, b_ref):
    ab = jnp.dot(x_ref[...], w_ref[...], preferred_element_type=jnp.float32)
    a_ref[...] = ab[:, :D]
    b_ref[...] = ab[:, D:]


def _precompute(x, w_ab):
    return pl.pallas_call(
        _pre_body,
        out_shape=(jax.ShapeDtypeStruct((N_NODES, D), jnp.float32),
                   jax.ShapeDtypeStruct((N_NODES, D), jnp.float32)),
    )(x, w_ab)


def _gather_body(a_hbm, b_hbm, packed_idx, ga_hbm, gb_hbm,
                 idxb, ra, rb, sem_a, sem_b, sem_wa, sem_wb):
    c = lax.axis_index("c")
    s = lax.axis_index("s")
    wid = s * NC + c

    pltpu.sync_copy(packed_idx.at[wid * NGRP], idxb.at[0])

    def gather_descs(j, slot):
        g2 = j // IDXG
        jj2 = j % IDXG
        da = pltpu.make_async_copy(
            a_hbm.at[idxb.at[g2 % 2, 2 * jj2, pl.ds(0, K)]],
            ra.at[slot], sem_a.at[slot])
        db = pltpu.make_async_copy(
            b_hbm.at[idxb.at[g2 % 2, 2 * jj2 + 1, pl.ds(0, K)]],
            rb.at[slot], sem_b.at[slot])
        return da, db

    def issue(j, slot):
        da, db = gather_descs(j, slot)
        da.start()
        db.start()

    def drain_writes(slot):
        pltpu.make_async_copy(ra.at[slot], ga_hbm.at[pl.ds(0, K)],
                              sem_wa.at[slot]).wait()
        pltpu.make_async_copy(rb.at[slot], gb_hbm.at[pl.ds(0, K)],
                              sem_wb.at[slot]).wait()

    issue(0, 0)
    issue(1, 1)

    def chunk(j, carry):
        slot = j % RING
        g = j // IDXG
        jj = j % IDXG
        da, db = gather_descs(j, slot)
        da.wait()
        db.wait()
        off = pl.multiple_of(wid * EP + j * K, 8)
        pltpu.async_copy(ra.at[slot], ga_hbm.at[pl.ds(off, K)],
                         sem_wa.at[slot])
        pltpu.async_copy(rb.at[slot], gb_hbm.at[pl.ds(off, K)],
                         sem_wb.at[slot])

        @pl.when((jj == IDXG - 2) & (g + 1 < NGRP))
        def _stage_idx():
            pltpu.sync_copy(packed_idx.at[wid * NGRP + g + 1],
                            idxb.at[(g + 1) % 2])

        nxt = (j + 2) % RING

        @pl.when((j + 2 < CHUNKS) & (j >= 1))
        def _drain():
            drain_writes(nxt)

        @pl.when(j + 2 < CHUNKS)
        def _prefetch():
            issue(j + 2, nxt)

        return carry

    lax.fori_loop(0, CHUNKS, chunk, 0, unroll=False)
    for jj in range(CHUNKS - RING, CHUNKS):
        drain_writes(jj % RING)


def _gather_pair(a, b, packed_idx):
    return pl.kernel(
        _gather_body,
        out_type=(jax.ShapeDtypeStruct((N_EDGES, D), jnp.float32),
                  jax.ShapeDtypeStruct((N_EDGES, D), jnp.float32)),
        mesh=_mesh(),
        scratch_types=[
            pltpu.VMEM((2, 2 * IDXG, 128), jnp.int32),
            pltpu.VMEM((RING, K, D), jnp.float32),
            pltpu.VMEM((RING, K, D), jnp.float32),
            pltpu.SemaphoreType.DMA((RING,)),
            pltpu.SemaphoreType.DMA((RING,)),
            pltpu.SemaphoreType.DMA((RING,)),
            pltpu.SemaphoreType.DMA((RING,)),
        ],
    )(a, b, packed_idx)


def _edge_mlp_body(ga_ref, gb_ref, ea_ref, w1c_ref, b1_ref, w2_ref, b2_ref,
                   m_ref):
    z = (ga_ref[...] + gb_ref[...]
         + jnp.dot(ea_ref[...], w1c_ref[...], preferred_element_type=jnp.float32)
         + b1_ref[...])
    h = jnp.maximum(z, 0.0).astype(jnp.bfloat16)
    m_ref[...] = (jnp.dot(h, w2_ref[...], preferred_element_type=jnp.float32)
                  + b2_ref[...])


def _edge_mlp(ga, gb, edge_attr, w1c, b1, w2, b2, block_e=4000):
    ne = ga.shape[0]
    grid = ne // block_e
    return pl.pallas_call(
        _edge_mlp_body,
        grid=(grid,),
        in_specs=[
            pl.BlockSpec((block_e, D), lambda i: (i, 0)),
            pl.BlockSpec((block_e, D), lambda i: (i, 0)),
            pl.BlockSpec((block_e, ED), lambda i: (i, 0)),
            pl.BlockSpec((ED, D), lambda i: (0, 0)),
            pl.BlockSpec((1, D), lambda i: (0, 0)),
            pl.BlockSpec((D, D), lambda i: (0, 0)),
            pl.BlockSpec((1, D), lambda i: (0, 0)),
        ],
        out_specs=pl.BlockSpec((block_e, D), lambda i: (i, 0)),
        out_shape=jax.ShapeDtypeStruct((ne, D), jnp.float32),
    )(ga, gb, edge_attr, w1c, b1, w2, b2)


def _scatter_body(m_hbm, row3d, part_hbm, idx_r, mb, agg, sem_l):
    c = lax.axis_index("c")
    s = lax.axis_index("s")
    wid = s * NC + c

    def zrow(e, carry):
        for v in range(D // 16):
            mb[0, e, pl.ds(v * 16, 16)] = jnp.zeros((16,), jnp.float32)
        return carry

    lax.fori_loop(0, K, zrow, 0, unroll=False)
    for t in range(7):
        zoff = pl.multiple_of(s * SUB_ROWS + t * K, 8)
        pltpu.sync_copy(mb.at[0], agg.at[pl.ds(zoff, K)])
    zoff = pl.multiple_of(s * SUB_ROWS + 7 * K, 8)
    pltpu.sync_copy(mb.at[0, pl.ds(0, 64)], agg.at[pl.ds(zoff, 64)])

    @pl.when(s == NS - 1)
    def _zero_tail():
        pltpu.sync_copy(mb.at[0, pl.ds(0, TAIL_ROWS)],
                        agg.at[pl.ds(NS * SUB_ROWS, TAIL_ROWS)])

    plsc.subcore_barrier()

    pltpu.sync_copy(row3d.at[wid], idx_r)

    def load(j, slot):
        off = pl.multiple_of(wid * EP + j * K, 8)
        pltpu.async_copy(m_hbm.at[pl.ds(off, K)], mb.at[slot], sem_l.at[slot])

    load(0, 0)

    def chunk(j, carry):
        slot = j % 2
        pltpu.make_async_copy(m_hbm.at[pl.ds(0, K)], mb.at[slot],
                              sem_l.at[slot]).wait()

        @pl.when(j + 1 < CHUNKS)
        def _prefetch():
            load(j + 1, (j + 1) % 2)

        pltpu.sync_copy(mb.at[slot], agg.at[idx_r.at[j]], add=True)
        return carry

    lax.fori_loop(0, CHUNKS, chunk, 0, unroll=False)
    plsc.subcore_barrier()

    woff = pl.multiple_of(s * SUB_ROWS, 8)
    pltpu.sync_copy(agg.at[pl.ds(woff, SUB_ROWS)],
                    part_hbm.at[c, pl.ds(woff, SUB_ROWS)])

    @pl.when(s == NS - 1)
    def _write_tail():
        pltpu.sync_copy(agg.at[pl.ds(NS * SUB_ROWS, TAIL_ROWS)],
                        part_hbm.at[c, pl.ds(NS * SUB_ROWS, TAIL_ROWS)])


def _scatter_add(m, row3d):
    return pl.kernel(
        _scatter_body,
        out_type=jax.ShapeDtypeStruct((NC, N_NODES, D), jnp.float32),
        mesh=_mesh(),
        scratch_types=[
            pltpu.VMEM((CHUNKS, K), jnp.int32),
            pltpu.VMEM((2, K, D), jnp.float32),
            pltpu.VMEM_SHARED((N_NODES, D), jnp.float32),
            pltpu.SemaphoreType.DMA((2,)),
        ],
    )(m, row3d)


def _node_mlp_body(x_ref, p_ref, wnx_ref, wna_ref, bn1_ref, wn2_ref, bn2_ref,
                   o_ref):
    p = p_ref[...]
    agg = p[0] + p[1]
    t = (jnp.dot(x_ref[...], wnx_ref[...], preferred_element_type=jnp.float32)
         + jnp.dot(agg, wna_ref[...], preferred_element_type=jnp.float32)
         + bn1_ref[...])
    h = jnp.maximum(t, 0.0)
    o_ref[...] = (jnp.dot(h, wn2_ref[...], preferred_element_type=jnp.float32)
                  + bn2_ref[...])


def _node_mlp(x, parts, wnx, wna, bn1, wn2, bn2):
    return pl.pallas_call(
        _node_mlp_body,
        out_shape=jax.ShapeDtypeStruct((N_NODES, D), jnp.float32),
    )(x, parts, wnx, wna, bn1, wn2, bn2)


def kernel(x, edge_index, edge_attr, W_e1, b_e1, W_e2, b_e2,
           W_n1, b_n1, W_n2, b_n2):
    row = edge_index[0].astype(jnp.int32)
    col = edge_index[1].astype(jnp.int32)
    row3d = row.reshape(NW, CHUNKS, K)
    zpad = jnp.zeros((NW, CHUNKS, 128 - K), jnp.int32)
    rrows = jnp.concatenate([row.reshape(NW, CHUNKS, K), zpad], axis=2)
    crows = jnp.concatenate([col.reshape(NW, CHUNKS, K), zpad], axis=2)
    packed_idx = jnp.stack([rrows, crows], axis=2).reshape(
        NW * NGRP, 2 * IDXG, 128)

    w_ab = jnp.concatenate([W_e1[:D], W_e1[D:2 * D]], axis=1)
    a, b = _precompute(x, w_ab)

    ga, gb = _gather_pair(a, b, packed_idx)

    m = _edge_mlp(ga, gb, edge_attr, W_e1[2 * D:], b_e1.reshape(1, D),
                  W_e2.astype(jnp.bfloat16), b_e2.reshape(1, D))

    parts = _scatter_add(m, row3d)

    out = _node_mlp(x, parts, W_n1[:D], W_n1[D:], b_n1.reshape(1, D),
                    W_n2, b_n2.reshape(1, D))
    return out
